# baseline (speedup 1.0000x reference)
import jax
import jax.numpy as jnp
from jax import lax
from jax.experimental import pallas as pl
from jax.experimental.pallas import tpu as pltpu
from jax.experimental.pallas import tpu_sc as plsc

_C = 8
_THR = 0.8
_NC = 2
_NS = 16
_L = 16


def _proj_body(w_ref, x_ref, p_ref):
    x = x_ref[0]
    w = w_ref[...]
    p_ref[0] = jax.lax.dot_general(
        x, w, (((1,), (0,)), ((), ())),
        preferred_element_type=jnp.float32)


def _sc_body(proj_hbm, lab_hbm, bias_hbm, feats_hbm, scores_hbm,
             labv, pv, packv, mergev, bufv, outv, shv):
    n = 4096
    chunk_sz = n // 4
    ngrp = chunk_sz // _L
    cidx = lax.axis_index("c")
    sidx = lax.axis_index("s")
    bag = cidx * 4 + sidx // 4
    chunk = sidx % 4
    base = bag * n + chunk * chunk_sz

    pltpu.sync_copy(lab_hbm.at[pl.ds(base, chunk_sz)], labv)
    pltpu.sync_copy(proj_hbm.at[pl.ds(base * 2, chunk_sz * 2)], pv)
    pltpu.sync_copy(bias_hbm, bufv.at[pl.ds(0, _L)])

    lane = lax.iota(jnp.int32, _L)
    stride2 = lane * 2
    zero = jnp.zeros((_L,), jnp.float32)

    def body(g, carry):
        s0, s1, cn = carry
        labg = labv[pl.ds(g * _L, _L)]
        i0 = stride2 + g * (2 * _L)
        p0 = plsc.load_gather(pv, [i0])
        p1 = plsc.load_gather(pv, [i0 + 1])
        s0o, s1o, cno = [], [], []
        for c in range(_C):
            m = labg == c
            s0o.append(s0[c] + jnp.where(m, p0, 0.0))
            s1o.append(s1[c] + jnp.where(m, p1, 0.0))
            cno.append(cn[c] + jnp.where(m, 1.0, 0.0))
        return tuple(s0o), tuple(s1o), tuple(cno)

    init = (tuple(zero for _ in range(_C)),) * 3
    s0, s1, cn = lax.fori_loop(0, ngrp, body, init)

    bv = bufv[pl.ds(0, _L)]
    b0 = bv[0]
    b1 = bv[1]
    pack_lo = zero
    pack_hi = zero
    for c in range(_C):
        pack_lo = jnp.where(lane == 2 * c, jnp.sum(s0[c]), pack_lo)
        pack_lo = jnp.where(lane == 2 * c + 1, jnp.sum(s1[c]), pack_lo)
        pack_hi = jnp.where(lane == c, jnp.sum(cn[c]), pack_hi)
    packv[pl.ds(0, _L)] = pack_lo
    packv[pl.ds(_L, _L)] = pack_hi

    pltpu.sync_copy(packv, shv.at[pl.ds(sidx * 32, 32)])
    plsc.subcore_barrier()

    @pl.when(chunk == 0)
    def _leader():
        pltpu.sync_copy(shv.at[pl.ds(sidx * 32, 128)], mergev)
        t_sums = (mergev[pl.ds(0, _L)] + mergev[pl.ds(32, _L)]
                  + mergev[pl.ds(64, _L)] + mergev[pl.ds(96, _L)])
        t_cnt = (mergev[pl.ds(_L, _L)] + mergev[pl.ds(32 + _L, _L)]
                 + mergev[pl.ds(64 + _L, _L)] + mergev[pl.ds(96 + _L, _L)])
        bufv[pl.ds(0, _L)] = t_sums
        bufv[pl.ds(_L, _L)] = t_cnt
        s0v = plsc.load_gather(bufv, [stride2])
        s1v = plsc.load_gather(bufv, [stride2 + 1])
        cnt = jnp.maximum(t_cnt, 1.0)
        l0 = s0v / cnt + b0
        l1 = s1v / cnt + b1
        m = jnp.maximum(l0, l1)
        e0 = jnp.exp(l0 - m)
        e1 = jnp.exp(l1 - m)
        sc = e1 / (e0 + e1)
        valid = lane < _C
        scm = jnp.where(valid, sc, -1.0)
        scp = jnp.where(valid, sc, 2.0)
        mx = jnp.max(scm)
        mn = jnp.min(scp)
        idx_max = plsc.all_reduce_ffs(scm == mx)
        idx_min = plsc.all_reduce_ffs(scp == mn)
        sel = jnp.where(mx < _THR, idx_min, idx_max)
        neg = jnp.float32(-3.0e38)
        l0s = jnp.max(jnp.where(lane == sel, l0, neg))
        l1s = jnp.max(jnp.where(lane == sel, l1, neg))
        outv[...] = jnp.where(lane == 0, l0s,
                              jnp.where(lane == 1, l1s, 0.0))
        pltpu.sync_copy(outv, feats_hbm.at[pl.ds(bag * _L, _L)])
        outv[...] = jnp.where(valid, sc, 0.0)
        pltpu.sync_copy(outv, scores_hbm.at[pl.ds(bag * _L, _L)])


def kernel(inst_feat, cluster_labels, W_head, b_head):
    B, N, D = inst_feat.shape
    ncls = W_head.shape[1]

    proj = pl.pallas_call(
        _proj_body,
        grid=(B,),
        in_specs=[
            pl.BlockSpec((D, ncls), lambda b: (0, 0)),
            pl.BlockSpec((1, N, D), lambda b: (b, 0, 0)),
        ],
        out_specs=pl.BlockSpec((1, N, ncls), lambda b: (b, 0, 0)),
        out_shape=jax.ShapeDtypeStruct((B, N, ncls), jnp.float32),
    )(W_head, inst_feat)

    bias16 = jnp.pad(b_head, (0, _L - ncls)).astype(jnp.float32)
    chunk_sz = N // 4

    mesh = plsc.VectorSubcoreMesh(core_axis_name="c", subcore_axis_name="s")
    sc_call = pl.kernel(
        _sc_body,
        out_type=(
            jax.ShapeDtypeStruct((B * _L,), jnp.float32),
            jax.ShapeDtypeStruct((B * _L,), jnp.float32),
        ),
        mesh=mesh,
        compiler_params=pltpu.CompilerParams(needs_layout_passes=False),
        scratch_types=[
            pltpu.VMEM((chunk_sz,), jnp.int32),
            pltpu.VMEM((2 * chunk_sz,), jnp.float32),
            pltpu.VMEM((32,), jnp.float32),
            pltpu.VMEM((128,), jnp.float32),
            pltpu.VMEM((32,), jnp.float32),
            pltpu.VMEM((_L,), jnp.float32),
            pltpu.VMEM_SHARED((_NS * 32,), jnp.float32),
        ],
    )
    featsp, scoresp = sc_call(
        proj.reshape(-1), cluster_labels.reshape(-1), bias16)
    feats = featsp.reshape(B, _L)[:, :ncls]
    scores = scoresp.reshape(B, _L)[:, :_C]
    return feats, scores
